# Initial kernel scaffold; baseline (speedup 1.0000x reference)
import jax
import jax.numpy as jnp
from jax.experimental import pallas as pl


def kernel(feats, edge_index, W1, b1, g1, be1, W2, b2, g2, be2, g3, be3):
    raise NotImplementedError("write your pallas kernel here")


# speedup vs baseline: 1.1851x; 1.1851x over previous
import functools

import jax
import jax.numpy as jnp
from jax import lax
from jax.experimental import pallas as pl
from jax.experimental.pallas import tpu as pltpu
from jax.experimental.pallas import tpu_sc as plsc

N = 10000
E = 320000
D = 128

NC = 2
NS = 16
NW = NC * NS
ROWS = 320
NPAD = NW * ROWS

BLK = 2000
NBLK = E // BLK
CHUNKS = BLK // 16

NEG_INF = float("-inf")


def _sc_aggregate(feats, src, dst):
  mesh = plsc.VectorSubcoreMesh(core_axis_name="c", subcore_axis_name="s")

  @functools.partial(
      pl.kernel,
      out_type=jax.ShapeDtypeStruct((NPAD, D), jnp.float32),
      mesh=mesh,
      scratch_types=[
          pltpu.VMEM((ROWS + 1, D), jnp.float32),
          pltpu.VMEM((BLK,), jnp.int32),
          pltpu.VMEM((BLK,), jnp.int32),
          pltpu.VMEM((BLK + 16,), jnp.int32),
          pltpu.VMEM((BLK + 16,), jnp.int32),
          pltpu.VMEM((16, D), jnp.float32),
          pltpu.SemaphoreType.DMA,
      ],
      compiler_params=pltpu.CompilerParams(needs_layout_passes=False),
  )
  def sc_kernel(feats_hbm, src_hbm, dst_hbm, out_hbm,
                acc, sbuf, dbuf, lsrc, ldst, gbuf, sem):
    wid = lax.axis_index("s") * NC + lax.axis_index("c")
    lo = wid * ROWS
    hi = lo + ROWS

    @pl.loop(0, ROWS + 1)
    def _(r):
      for q in range(D // 16):
        acc[r, pl.ds(q * 16, 16)] = jnp.full((16,), NEG_INF, jnp.float32)

    @pl.loop(0, NBLK)
    def _(b):
      pltpu.sync_copy(src_hbm.at[pl.ds(b * BLK, BLK)], sbuf)
      pltpu.sync_copy(dst_hbm.at[pl.ds(b * BLK, BLK)], dbuf)

      def filt(c, cnt):
        d = dbuf[pl.ds(c * 16, 16)]
        m = (d >= lo) & (d < hi)
        s = sbuf[pl.ds(c * 16, 16)]
        plsc.store_compressed(lsrc.at[pl.ds(cnt, 16)], s, mask=m)
        plsc.store_compressed(ldst.at[pl.ds(cnt, 16)], d - lo, mask=m)
        return cnt + jnp.sum(m.astype(jnp.int32))

      cnt = lax.fori_loop(0, CHUNKS, filt, jnp.int32(0))

      lsrc[pl.ds(cnt, 16)] = jnp.full((16,), 0, jnp.int32)
      ldst[pl.ds(cnt, 16)] = jnp.full((16,), ROWS, jnp.int32)
      ngroups = (cnt + 15) // 16

      def group(g, carry):
        idxv = lsrc[pl.ds(g * 16, 16)]
        dv = ldst[pl.ds(g * 16, 16)]
        pltpu.async_copy(feats_hbm.at[idxv], gbuf, sem).wait()
        for r in range(16):
          dl = dv[r]
          for q in range(D // 16):
            sl = pl.ds(q * 16, 16)
            acc[dl, sl] = jnp.maximum(acc[dl, sl], gbuf[r, sl])
        return carry

      lax.fori_loop(0, ngroups, group, jnp.int32(0))

    pltpu.sync_copy(acc.at[pl.ds(0, ROWS)], out_hbm.at[pl.ds(lo, ROWS)])

  return sc_kernel(feats, src, dst)


def _tc_mlp(feats, agg, w1t, b1, g1, be1, w2t, b2, g2, be2, g3, be3):
  def body(feats_ref, agg_ref, w1t_ref, b1_ref, g1_ref, be1_ref,
           w2t_ref, b2_ref, g2_ref, be2_ref, g3_ref, be3_ref, out_ref):
    a = agg_ref[...]
    a = jnp.where(a == NEG_INF, 0.0, a)
    h = feats_ref[...] + a

    def bn(x, gamma, beta):
      mu = jnp.mean(x, axis=0, keepdims=True)
      var = jnp.mean(x * x, axis=0, keepdims=True) - mu * mu
      return (x - mu) * lax.rsqrt(var + 1e-5) * gamma + beta

    h = jnp.dot(h, w1t_ref[...], preferred_element_type=jnp.float32)
    h = bn(h + b1_ref[...], g1_ref[...], be1_ref[...])
    h = jnp.maximum(h, 0.0)
    h = jnp.dot(h, w2t_ref[...], preferred_element_type=jnp.float32)
    h = bn(h + b2_ref[...], g2_ref[...], be2_ref[...])
    h = jnp.maximum(h, 0.0)
    out_ref[...] = bn(h, g3_ref[...], be3_ref[...])

  vecs = [b1, g1, be1, b2, g2, be2, g3, be3]
  vecs2d = [v.reshape(1, D) for v in vecs]
  return pl.pallas_call(
      body,
      out_shape=jax.ShapeDtypeStruct((N, D), jnp.float32),
  )(feats, agg, w1t, vecs2d[0], vecs2d[1], vecs2d[2],
    w2t, vecs2d[3], vecs2d[4], vecs2d[5], vecs2d[6], vecs2d[7])


@jax.jit
def kernel(feats, edge_index, W1, b1, g1, be1, W2, b2, g2, be2, g3, be3):
  src = edge_index[0]
  dst = edge_index[1]
  agg = _sc_aggregate(feats, src, dst)[:N]
  return _tc_mlp(feats, agg, W1.T, b1, g1, be1, W2.T, b2, g2, be2, g3, be3)


# speedup vs baseline: 1.4110x; 1.1907x over previous
import functools

import jax
import jax.numpy as jnp
from jax import lax
from jax.experimental import pallas as pl
from jax.experimental.pallas import tpu as pltpu
from jax.experimental.pallas import tpu_sc as plsc

N = 10000
E = 320000
D = 128

NC = 2
NS = 16
NW = NC * NS
ROWS = 320
NPAD = NW * ROWS

BLK = 2000
NBLK = E // BLK
CHUNKS = BLK // 16

NEG_INF = float("-inf")


def _sc_aggregate(feats, src, dst):
  mesh = plsc.VectorSubcoreMesh(core_axis_name="c", subcore_axis_name="s")

  @functools.partial(
      pl.kernel,
      out_type=jax.ShapeDtypeStruct((NPAD, D), jnp.float32),
      mesh=mesh,
      scratch_types=[
          pltpu.VMEM((ROWS + 1, D), jnp.float32),
          pltpu.VMEM((BLK,), jnp.int32),
          pltpu.VMEM((BLK,), jnp.int32),
          pltpu.VMEM((BLK + 32,), jnp.int32),
          pltpu.VMEM((BLK + 32,), jnp.int32),
          pltpu.VMEM((16, D), jnp.float32),
          pltpu.VMEM((16, D), jnp.float32),
          pltpu.SemaphoreType.DMA,
          pltpu.SemaphoreType.DMA,
      ],
      compiler_params=pltpu.CompilerParams(needs_layout_passes=False),
  )
  def sc_kernel(feats_hbm, src_hbm, dst_hbm, out_hbm,
                acc, sbuf, dbuf, lsrc, ldst, gbufa, gbufb, sema, semb):
    wid = lax.axis_index("s") * NC + lax.axis_index("c")
    lo = wid * ROWS
    hi = lo + ROWS

    @pl.loop(0, ROWS + 1)
    def _(r):
      for q in range(D // 16):
        acc[r, pl.ds(q * 16, 16)] = jnp.full((16,), NEG_INF, jnp.float32)

    @pl.loop(0, NBLK)
    def _(b):
      pltpu.sync_copy(src_hbm.at[pl.ds(b * BLK, BLK)], sbuf)
      pltpu.sync_copy(dst_hbm.at[pl.ds(b * BLK, BLK)], dbuf)

      def filt(c, cnt):
        d = dbuf[pl.ds(c * 16, 16)]
        m = (d >= lo) & (d < hi)
        nm = plsc.all_reduce_population_count(m)[0]

        def do_store(cc):
          s = sbuf[pl.ds(c * 16, 16)]
          plsc.store_compressed(lsrc.at[pl.ds(cc, 16)], s, mask=m)
          plsc.store_compressed(ldst.at[pl.ds(cc, 16)], d - lo, mask=m)
          return cc + nm

        return lax.cond(nm > 0, do_store, lambda cc: cc, cnt)

      cnt = lax.fori_loop(0, CHUNKS, filt, jnp.int32(0))

      pad_s = jnp.broadcast_to(lo, (16,))
      pad_d = jnp.full((16,), ROWS, jnp.int32)
      lsrc[pl.ds(cnt, 16)] = pad_s
      ldst[pl.ds(cnt, 16)] = pad_d
      lsrc[pl.ds(cnt + 16, 16)] = pad_s
      ldst[pl.ds(cnt + 16, 16)] = pad_d
      ngroups = (cnt + 15) // 16
      nit = (ngroups + 1) // 2

      def fire(gb, sm, g):
        idxv = lsrc[pl.ds(g * 16, 16)]
        pltpu.make_async_copy(feats_hbm.at[idxv], gb, sm).start()

      def drain(gb, sm, g):
        idxv = lsrc[pl.ds(g * 16, 16)]
        pltpu.make_async_copy(feats_hbm.at[idxv], gb, sm).wait()

      def process(gb, g):
        dv = ldst[pl.ds(g * 16, 16)]
        for r in range(16):
          dl = dv[r]
          for q in range(D // 16):
            sl = pl.ds(q * 16, 16)
            acc[dl, sl] = jnp.maximum(acc[dl, sl], gb[r, sl])

      @pl.when(nit > 0)
      def _():
        fire(gbufa, sema, 0)

      def group2(i, carry):
        g0 = 2 * i
        fire(gbufb, semb, g0 + 1)
        drain(gbufa, sema, g0)
        process(gbufa, g0)

        @pl.when(i + 1 < nit)
        def _():
          fire(gbufa, sema, g0 + 2)

        drain(gbufb, semb, g0 + 1)
        process(gbufb, g0 + 1)
        return carry

      lax.fori_loop(0, nit, group2, jnp.int32(0))

    pltpu.sync_copy(acc.at[pl.ds(0, ROWS)], out_hbm.at[pl.ds(lo, ROWS)])

  return sc_kernel(feats, src, dst)


def _tc_mlp(feats, agg, w1t, b1, g1, be1, w2t, b2, g2, be2, g3, be3):
  def body(feats_ref, agg_ref, w1t_ref, b1_ref, g1_ref, be1_ref,
           w2t_ref, b2_ref, g2_ref, be2_ref, g3_ref, be3_ref, out_ref):
    a = agg_ref[...]
    a = jnp.where(a == NEG_INF, 0.0, a)
    h = feats_ref[...] + a

    def bn(x, gamma, beta):
      mu = jnp.mean(x, axis=0, keepdims=True)
      var = jnp.mean(x * x, axis=0, keepdims=True) - mu * mu
      return (x - mu) * lax.rsqrt(var + 1e-5) * gamma + beta

    h = jnp.dot(h, w1t_ref[...], preferred_element_type=jnp.float32)
    h = bn(h + b1_ref[...], g1_ref[...], be1_ref[...])
    h = jnp.maximum(h, 0.0)
    h = jnp.dot(h, w2t_ref[...], preferred_element_type=jnp.float32)
    h = bn(h + b2_ref[...], g2_ref[...], be2_ref[...])
    h = jnp.maximum(h, 0.0)
    out_ref[...] = bn(h, g3_ref[...], be3_ref[...])

  vecs = [b1, g1, be1, b2, g2, be2, g3, be3]
  vecs2d = [v.reshape(1, D) for v in vecs]
  return pl.pallas_call(
      body,
      out_shape=jax.ShapeDtypeStruct((N, D), jnp.float32),
  )(feats, agg, w1t, vecs2d[0], vecs2d[1], vecs2d[2],
    w2t, vecs2d[3], vecs2d[4], vecs2d[5], vecs2d[6], vecs2d[7])


@jax.jit
def kernel(feats, edge_index, W1, b1, g1, be1, W2, b2, g2, be2, g3, be3):
  src = edge_index[0]
  dst = edge_index[1]
  agg = _sc_aggregate(feats, src, dst)[:N]
  return _tc_mlp(feats, agg, W1.T, b1, g1, be1, W2.T, b2, g2, be2, g3, be3)
